# baseline (speedup 1.0000x reference)
import functools

import jax
import jax.numpy as jnp
from jax import lax
from jax.experimental import pallas as pl
from jax.experimental.pallas import tpu as pltpu
from jax.experimental.pallas import tpu_sc as plsc

N_NODES = 10000
D = 128
N_EDGES = 320000

NC = 2
NS = 16
TILES = NC * NS
CHUNK = 128
CHUNKS_PER_TILE = 80
PER_TILE = CHUNK * CHUNKS_PER_TILE
TOT_EDGES = PER_TILE * TILES
NODES_PAD = 10240
HIST_ROWS = NODES_PAD // CHUNK
ROWS_PER_TILE = NODES_PAD // NS


def _sc_aggregate(src_hbm, dst_hbm, verts_hbm, agg_out,
                  src_v, dst_v, rows_a, rows_b, agg_sh, sem_a, sem_b):
    c = lax.axis_index("c")
    s = lax.axis_index("s")
    wid = s * NC + c

    zrow = jnp.zeros((16,), jnp.float32)
    def _zero_body(i, _):
        for k in range(D // 16):
            rows_a[i, pl.ds(16 * k, 16)] = zrow
        return 0
    lax.fori_loop(0, CHUNK, _zero_body, 0)
    for r in range(ROWS_PER_TILE // CHUNK):
        pltpu.sync_copy(rows_a, agg_sh.at[pl.ds(s * ROWS_PER_TILE + r * CHUNK, CHUNK)])

    plsc.subcore_barrier()

    half = CHUNKS_PER_TILE // 2

    def _pass_body(p, _):
        pltpu.sync_copy(src_hbm.at[wid * 2 + p], src_v)
        pltpu.sync_copy(dst_hbm.at[wid * 2 + p], dst_v)
        def _pair_body(i, _):
            ca = 2 * i
            cb = 2 * i + 1
            da = pltpu.async_copy(verts_hbm.at[src_v.at[ca]], rows_a, sem_a)
            db = pltpu.async_copy(verts_hbm.at[src_v.at[cb]], rows_b, sem_b)
            da.wait()
            pltpu.sync_copy(rows_a, agg_sh.at[dst_v.at[ca]], add=True)
            db.wait()
            pltpu.sync_copy(rows_b, agg_sh.at[dst_v.at[cb]], add=True)
            return 0
        lax.fori_loop(0, half // 2, _pair_body, 0)
        return 0
    lax.fori_loop(0, 2, _pass_body, 0)

    plsc.subcore_barrier()

    rbase = s * ROWS_PER_TILE
    pltpu.sync_copy(agg_sh.at[pl.ds(rbase, ROWS_PER_TILE)],
                    agg_out.at[c].at[pl.ds(rbase, ROWS_PER_TILE)])


def _tc_deghist(dstc, o):
    i = pl.program_id(0)
    d = dstc[...]
    hi = jnp.floor(d * (1.0 / CHUNK))
    lo = d - hi * CHUNK
    ihi = lax.broadcasted_iota(jnp.int32, (1, HIST_ROWS), 1).astype(jnp.float32)
    ilo = lax.broadcasted_iota(jnp.int32, (1, CHUNK), 1).astype(jnp.float32)
    oh_hi = (hi == ihi).astype(jnp.bfloat16)
    oh_lo = (lo == ilo).astype(jnp.bfloat16)
    acc = lax.dot_general(oh_hi, oh_lo, (((0,), (0,)), ((), ())),
                          preferred_element_type=jnp.float32)
    @pl.when(i == 0)
    def _():
        o[...] = acc
    @pl.when(i > 0)
    def _():
        o[...] = o[...] + acc


def _tc_degnorm(dg, o):
    o[...] = lax.rsqrt(dg[...] + 1.0)


def _tc_finish(p0, p1, v, nm, wt, bb, o):
    x = (p0[...] + p1[...] + v[...]) * nm[...]
    o[...] = jnp.dot(x, wt[...], preferred_element_type=jnp.float32) + bb[...]


def kernel(vertices, edges, W, b):
    pad = TOT_EDGES - N_EDGES
    src = jnp.concatenate([edges[1], jnp.zeros((pad,), jnp.int32)])
    dst = jnp.concatenate([edges[0], jnp.full((pad,), N_NODES, jnp.int32)])
    src2 = src.reshape(TILES * 2, CHUNKS_PER_TILE // 2, CHUNK)
    dst2 = dst.reshape(TILES * 2, CHUNKS_PER_TILE // 2, CHUNK)
    dst = dst.reshape(TILES, CHUNKS_PER_TILE, CHUNK)

    mesh = plsc.VectorSubcoreMesh(core_axis_name="c", subcore_axis_name="s")

    agg = functools.partial(
        pl.kernel,
        mesh=mesh,
        out_type=jax.ShapeDtypeStruct((NC, NODES_PAD, D), jnp.float32),
        scratch_types=[
            pltpu.VMEM((CHUNKS_PER_TILE // 2, CHUNK), jnp.int32),
            pltpu.VMEM((CHUNKS_PER_TILE // 2, CHUNK), jnp.int32),
            pltpu.VMEM((CHUNK, D), jnp.float32),
            pltpu.VMEM((CHUNK, D), jnp.float32),
            pltpu.VMEM_SHARED((NODES_PAD, D), jnp.float32),
            pltpu.SemaphoreType.DMA,
            pltpu.SemaphoreType.DMA,
        ],
    )(_sc_aggregate)(src2, dst2, vertices)

    eblk = 8192
    dstc = dst.reshape(TOT_EDGES, 1).astype(jnp.float32)
    deg2d = pl.pallas_call(
        _tc_deghist,
        grid=(TOT_EDGES // eblk,),
        in_specs=[pl.BlockSpec((eblk, 1), lambda i: (i, 0))],
        out_specs=pl.BlockSpec((HIST_ROWS, CHUNK), lambda i: (0, 0)),
        out_shape=jax.ShapeDtypeStruct((HIST_ROWS, CHUNK), jnp.float32),
    )(dstc)

    rows_blk = 1024
    grid = (NODES_PAD // rows_blk,)

    norm = pl.pallas_call(
        _tc_degnorm,
        out_shape=jax.ShapeDtypeStruct((HIST_ROWS, CHUNK), jnp.float32),
    )(deg2d)
    normb = jnp.broadcast_to(norm.reshape(NODES_PAD, 1), (NODES_PAD, D))

    vpad = jnp.pad(vertices, ((0, NODES_PAD - N_NODES), (0, 0)))
    out = pl.pallas_call(
        _tc_finish,
        grid=grid,
        in_specs=[
            pl.BlockSpec((rows_blk, D), lambda i: (i, 0)),
            pl.BlockSpec((rows_blk, D), lambda i: (i, 0)),
            pl.BlockSpec((rows_blk, D), lambda i: (i, 0)),
            pl.BlockSpec((rows_blk, D), lambda i: (i, 0)),
            pl.BlockSpec((D, D), lambda i: (0, 0)),
            pl.BlockSpec((1, D), lambda i: (0, 0)),
        ],
        out_specs=pl.BlockSpec((rows_blk, D), lambda i: (i, 0)),
        out_shape=jax.ShapeDtypeStruct((NODES_PAD, D), jnp.float32),
    )(
        agg[0], agg[1], vpad, normb,
        W.T, b.reshape(1, D),
    )
    return out[:N_NODES]
